# baseline (speedup 1.0000x reference)
import dataclasses
import functools

import jax
import jax.numpy as jnp
from jax import lax
from jax.experimental import pallas as pl
from jax.experimental.pallas import tpu as pltpu
from jax.experimental.pallas import tpu_sc as plsc

NC = 2
NS = 16
NW = NC * NS
L = 16
WIN = 128
CHUNK_ROWS = 16

_mesh = lambda: plsc.VectorSubcoreMesh(core_axis_name="c", subcore_axis_name="s")


def _sc_params():
    cp = pltpu.CompilerParams()
    fields = pltpu.CompilerParams.__dataclass_fields__
    if "needs_layout_passes" in fields:
        cp = dataclasses.replace(cp, needs_layout_passes=False)
    if "use_tc_tiling_on_sc" in fields:
        cp = dataclasses.replace(cp, use_tc_tiling_on_sc=False)
    return cp


def _sc_deg(npad, rows_w):
    span = npad // NS
    nchunks = rows_w // CHUNK_ROWS

    @functools.partial(
        pl.kernel,
        out_type=[jax.ShapeDtypeStruct((npad,), jnp.float32)] * NC,
        mesh=_mesh(),
        scratch_types=[
            pltpu.VMEM((2, CHUNK_ROWS, WIN), jnp.int32),
            pltpu.VMEM((2, CHUNK_ROWS, WIN), jnp.float32),
            pltpu.VMEM((span,), jnp.float32),
            pltpu.VMEM_SHARED((npad,), jnp.float32),
            pltpu.SemaphoreType.DMA((2, 2)),
            pltpu.SemaphoreType.DMA((4,)),
        ],
        compiler_params=_sc_params(),
    )
    def k(dst_hbm, ew_hbm, z_hbm, out0_hbm, out1_hbm, dst_v, ew_v, stg_v,
          acc_sh, csem, ssem):
        c = lax.axis_index("c")
        s = lax.axis_index("s")
        wid = s * NC + c
        sp = pl.ds(s * span, span)
        pltpu.sync_copy(z_hbm.at[sp], stg_v)
        pltpu.sync_copy(stg_v, acc_sh.at[sp])
        plsc.subcore_barrier()
        base = wid * rows_w

        def issue_chunk(cc, b):
            ro = pl.ds(base + cc * CHUNK_ROWS, CHUNK_ROWS)
            pltpu.async_copy(dst_hbm.at[ro], dst_v.at[b], csem.at[0, b])
            pltpu.async_copy(ew_hbm.at[ro], ew_v.at[b], csem.at[1, b])

        def wait_chunk(b):
            ro = pl.ds(base, CHUNK_ROWS)
            pltpu.make_async_copy(dst_hbm.at[ro], dst_v.at[b],
                                  csem.at[0, b]).wait()
            pltpu.make_async_copy(ew_hbm.at[ro], ew_v.at[b],
                                  csem.at[1, b]).wait()

        def process_chunk(cc, b):
            wait_chunk(b)
            scatters = {}
            for j in range(CHUNK_ROWS):
                if j - 4 in scatters:
                    scatters[j - 4].wait()
                scatters[j] = pltpu.async_copy(
                    ew_v.at[b, j], acc_sh.at[dst_v.at[b, j]], ssem.at[j % 4],
                    add=True)
            for j in range(CHUNK_ROWS - 4, CHUNK_ROWS):
                scatters[j].wait()

            @pl.when(cc + 2 < nchunks)
            def _():
                issue_chunk(cc + 2, b)

        issue_chunk(0, 0)
        issue_chunk(1, 1)

        @pl.loop(0, nchunks - 1, step=2)
        def _(cc):
            process_chunk(cc, 0)
            process_chunk(cc + 1, 1)

        process_chunk(nchunks - 1, 0)

        plsc.subcore_barrier()
        pltpu.sync_copy(acc_sh.at[sp], stg_v)

        @pl.when(c == 0)
        def _():
            pltpu.sync_copy(stg_v, out0_hbm.at[sp])

        @pl.when(c == 1)
        def _():
            pltpu.sync_copy(stg_v, out1_hbm.at[sp])

    return k


def _sc_edge16(npad, rows_w):
    span = npad // NS
    nchunks = rows_w // CHUNK_ROWS

    @functools.partial(
        pl.kernel,
        out_type=[jax.ShapeDtypeStruct((npad, L), jnp.float32)] * NC,
        mesh=_mesh(),
        scratch_types=[
            pltpu.VMEM((2, CHUNK_ROWS, WIN), jnp.int32),
            pltpu.VMEM((2, CHUNK_ROWS, WIN), jnp.int32),
            pltpu.VMEM((2, CHUNK_ROWS, WIN), jnp.float32),
            pltpu.VMEM((2, WIN, L), jnp.float32),
            pltpu.VMEM((2, WIN, L), jnp.float32),
            pltpu.VMEM_SHARED((npad, L), jnp.float32),
            pltpu.SemaphoreType.DMA((3, 2)),
            pltpu.SemaphoreType.DMA((2,)),
            pltpu.SemaphoreType.DMA((2,)),
        ],
        compiler_params=_sc_params(),
    )
    def k(src_hbm, dst_hbm, ew_hbm, g1_hbm, out0_hbm, out1_hbm,
          src_v, dst_v, ew_v, rows_v, pay_v, acc_sh, csem, gsem, ssem):
        c = lax.axis_index("c")
        s = lax.axis_index("s")
        wid = s * NC + c

        @pl.loop(0, WIN)
        def _(i):
            pay_v[0, i, :] = jnp.zeros((L,), jnp.float32)

        @pl.loop(0, span, step=WIN)
        def _(r):
            pltpu.sync_copy(pay_v.at[0], acc_sh.at[pl.ds(s * span + r, WIN)])

        plsc.subcore_barrier()
        base = wid * rows_w

        def issue_chunk(cc, b):
            ro = pl.ds(base + cc * CHUNK_ROWS, CHUNK_ROWS)
            pltpu.async_copy(src_hbm.at[ro], src_v.at[b], csem.at[0, b])
            pltpu.async_copy(dst_hbm.at[ro], dst_v.at[b], csem.at[1, b])
            pltpu.async_copy(ew_hbm.at[ro], ew_v.at[b], csem.at[2, b])

        def wait_chunk(b):
            ro = pl.ds(base, CHUNK_ROWS)
            pltpu.make_async_copy(src_hbm.at[ro], src_v.at[b],
                                  csem.at[0, b]).wait()
            pltpu.make_async_copy(dst_hbm.at[ro], dst_v.at[b],
                                  csem.at[1, b]).wait()
            pltpu.make_async_copy(ew_hbm.at[ro], ew_v.at[b],
                                  csem.at[2, b]).wait()

        def scale_window(b, j, jb):
            @pl.loop(0, WIN // L, step=2)
            def _(t):
                for t2 in range(2):
                    col = pl.multiple_of((t + t2) * L, L)
                    w16 = ew_v[b, j, pl.ds(col, L)]
                    for u in range(L):
                        i = (t + t2) * L + u
                        wv = jnp.full((L,), w16[u])
                        pay_v[jb, i, :] = rows_v[jb, i, :] * wv

        def process_chunk(cc, b):
            wait_chunk(b)
            gathers = {}
            scatters = {}
            gathers[0] = pltpu.async_copy(
                g1_hbm.at[src_v.at[b, 0]], rows_v.at[0], gsem.at[0])
            for j in range(CHUNK_ROWS):
                jb = j % 2
                if j + 1 < CHUNK_ROWS:
                    gathers[j + 1] = pltpu.async_copy(
                        g1_hbm.at[src_v.at[b, j + 1]], rows_v.at[(j + 1) % 2],
                        gsem.at[(j + 1) % 2])
                gathers[j].wait()
                if j - 2 in scatters:
                    scatters[j - 2].wait()
                scale_window(b, j, jb)
                scatters[j] = pltpu.async_copy(
                    pay_v.at[jb], acc_sh.at[dst_v.at[b, j]], ssem.at[jb],
                    add=True)
            scatters[CHUNK_ROWS - 2].wait()
            scatters[CHUNK_ROWS - 1].wait()

            @pl.when(cc + 2 < nchunks)
            def _():
                issue_chunk(cc + 2, b)

        issue_chunk(0, 0)
        issue_chunk(1, 1)

        @pl.loop(0, nchunks - 1, step=2)
        def _(cc):
            process_chunk(cc, 0)
            process_chunk(cc + 1, 1)

        process_chunk(nchunks - 1, 0)

        plsc.subcore_barrier()

        @pl.loop(0, span, step=WIN)
        def _(r):
            pltpu.sync_copy(acc_sh.at[pl.ds(s * span + r, WIN)], rows_v.at[0])

            @pl.when(c == 0)
            def _():
                pltpu.sync_copy(rows_v.at[0],
                                out0_hbm.at[pl.ds(s * span + r, WIN)])

            @pl.when(c == 1)
            def _():
                pltpu.sync_copy(rows_v.at[0],
                                out1_hbm.at[pl.ds(s * span + r, WIN)])

    return k


def _sc_edge1(npad, rows_w):
    span = npad // NS

    nchunks = rows_w // CHUNK_ROWS

    @functools.partial(
        pl.kernel,
        out_type=[jax.ShapeDtypeStruct((npad,), jnp.float32)] * NC,
        mesh=_mesh(),
        scratch_types=[
            pltpu.VMEM((2, CHUNK_ROWS, WIN), jnp.int32),
            pltpu.VMEM((2, CHUNK_ROWS, WIN), jnp.int32),
            pltpu.VMEM((2, CHUNK_ROWS, WIN), jnp.float32),
            pltpu.VMEM((4, WIN), jnp.float32),
            pltpu.VMEM((npad,), jnp.float32),
            pltpu.VMEM((span,), jnp.float32),
            pltpu.VMEM_SHARED((npad,), jnp.float32),
            pltpu.SemaphoreType.DMA((3, 2)),
            pltpu.SemaphoreType.DMA((4,)),
        ],
        compiler_params=_sc_params(),
    )
    def k(src_hbm, dst_hbm, ew_hbm, g2_hbm, z_hbm, out0_hbm, out1_hbm,
          src_v, dst_v, ew_v, pay_v, g2_v, stg_v, acc_sh, csem, ssem):
        c = lax.axis_index("c")
        s = lax.axis_index("s")
        wid = s * NC + c
        sp = pl.ds(s * span, span)
        pltpu.sync_copy(z_hbm.at[sp], stg_v)
        pltpu.sync_copy(stg_v, acc_sh.at[sp])
        pltpu.sync_copy(g2_hbm, g2_v)
        plsc.subcore_barrier()
        base = wid * rows_w

        def issue_chunk(cc, b):
            ro = pl.ds(base + cc * CHUNK_ROWS, CHUNK_ROWS)
            pltpu.async_copy(src_hbm.at[ro], src_v.at[b], csem.at[0, b])
            pltpu.async_copy(dst_hbm.at[ro], dst_v.at[b], csem.at[1, b])
            pltpu.async_copy(ew_hbm.at[ro], ew_v.at[b], csem.at[2, b])

        def wait_chunk(b):
            ro = pl.ds(base, CHUNK_ROWS)
            pltpu.make_async_copy(src_hbm.at[ro], src_v.at[b],
                                  csem.at[0, b]).wait()
            pltpu.make_async_copy(dst_hbm.at[ro], dst_v.at[b],
                                  csem.at[1, b]).wait()
            pltpu.make_async_copy(ew_hbm.at[ro], ew_v.at[b],
                                  csem.at[2, b]).wait()

        def process_chunk(cc, b):
            wait_chunk(b)
            scatters = {}
            for j in range(CHUNK_ROWS):
                jb = j % 4
                if j - 4 in scatters:
                    scatters[j - 4].wait()

                @pl.loop(0, WIN // L)
                def _(t):
                    col = pl.multiple_of(t * L, L)
                    idx16 = src_v[b, j, pl.ds(col, L)]
                    w16 = ew_v[b, j, pl.ds(col, L)]
                    vals = plsc.load_gather(g2_v, [idx16])
                    pay_v[jb, pl.ds(col, L)] = vals * w16

                scatters[j] = pltpu.async_copy(
                    pay_v.at[jb], acc_sh.at[dst_v.at[b, j]], ssem.at[jb],
                    add=True)
            for j in range(CHUNK_ROWS - 4, CHUNK_ROWS):
                scatters[j].wait()

            @pl.when(cc + 2 < nchunks)
            def _():
                issue_chunk(cc + 2, b)

        issue_chunk(0, 0)
        issue_chunk(1, 1)

        @pl.loop(0, nchunks - 1, step=2)
        def _(cc):
            process_chunk(cc, 0)
            process_chunk(cc + 1, 1)

        process_chunk(nchunks - 1, 0)

        plsc.subcore_barrier()
        pltpu.sync_copy(acc_sh.at[sp], stg_v)

        @pl.when(c == 0)
        def _():
            pltpu.sync_copy(stg_v, out0_hbm.at[sp])

        @pl.when(c == 1)
        def _():
            pltpu.sync_copy(stg_v, out1_hbm.at[sp])

    return k


_BLK = 2048


def _row_spec(cols):
    return pl.BlockSpec((_BLK, cols), lambda i: (i, 0))


def _full_spec(shape):
    return pl.BlockSpec(shape, lambda i: tuple(0 for _ in shape))


def _tc_h(xpack, wbig):
    rows = xpack.shape[0]
    blk = _BLK // 8

    def body(x_ref, w_ref, h_ref):
        h_ref[...] = jnp.dot(x_ref[...], w_ref[...],
                             preferred_element_type=jnp.float32)

    return pl.pallas_call(
        body,
        grid=(rows // blk,),
        in_specs=[pl.BlockSpec((blk, 200), lambda i: (i, 0)),
                  _full_spec((200, 128))],
        out_specs=pl.BlockSpec((blk, 128), lambda i: (i, 0)),
        out_shape=jax.ShapeDtypeStruct((rows, 128), jnp.float32),
    )(xpack, wbig)


def _rsqrt16(x):
    i = plsc.bitcast(x, jnp.int32)
    i = jnp.int32(0x5F3759DF) - (i >> 1)
    y = plsc.bitcast(i, jnp.float32)
    for _ in range(3):
        y = y * (1.5 - 0.5 * x * y * y)
    return y


def _sc_a2(npad):
    span = npad // NW
    nwin = span // WIN

    @functools.partial(
        pl.kernel,
        out_type=[jax.ShapeDtypeStruct((npad, L), jnp.float32),
                  jax.ShapeDtypeStruct((npad,), jnp.float32)],
        mesh=_mesh(),
        scratch_types=[
            pltpu.VMEM((span,), jnp.float32),
            pltpu.VMEM((span,), jnp.float32),
            pltpu.VMEM((span,), jnp.float32),
            pltpu.VMEM((L, WIN), jnp.float32),
            pltpu.VMEM((WIN, L), jnp.float32),
        ],
        compiler_params=_sc_params(),
    )
    def k(d0_hbm, d1_hbm, h_hbm, g1_hbm, dinv_hbm, d0_v, d1_v, dv_v, hw_v,
          gw_v):
        c = lax.axis_index("c")
        s = lax.axis_index("s")
        wid = s * NC + c
        nb = wid * span
        pltpu.sync_copy(d0_hbm.at[pl.ds(nb, span)], d0_v)
        pltpu.sync_copy(d1_hbm.at[pl.ds(nb, span)], d1_v)

        @pl.loop(0, span, step=L)
        def _(i):
            d = d0_v[pl.ds(i, L)] + d1_v[pl.ds(i, L)] + 1.0
            dv_v[pl.ds(i, L)] = _rsqrt16(d)

        pltpu.sync_copy(dv_v, dinv_hbm.at[pl.ds(nb, span)])

        @pl.loop(0, nwin)
        def _(w):
            pltpu.sync_copy(h_hbm.at[pl.ds(wid * span // 8 + w * L, L)], hw_v)
            for g in range(WIN // L):
                dw = dv_v[pl.ds(w * WIN + g * L, L)]
                for u in range(L):
                    kk = g * L + u
                    gw_v[kk, :] = hw_v[kk // 8, pl.ds(16 * (kk % 8), L)] \
                        * jnp.full((L,), dw[u])
            pltpu.sync_copy(gw_v, g1_hbm.at[pl.ds(nb + w * WIN, WIN)])

    return k


_GATHER_DNUMS = lax.GatherDimensionNumbers(
    offset_dims=(), collapsed_slice_dims=(0,), start_index_map=(0,))


def _lane_perm(v, idx):
    return lax.gather(v, idx[:, None], _GATHER_DNUMS, slice_sizes=(1,),
                      mode=lax.GatherScatterMode.PROMISE_IN_BOUNDS)


def _lane_sum(v):
    lanes = lax.iota(jnp.int32, L)
    for k in (1, 2, 4, 8):
        v = v + _lane_perm(v, lanes ^ k)
    return v


def _sc_b(npad):
    span = npad // NW
    nwin = span // WIN

    @functools.partial(
        pl.kernel,
        out_type=[jax.ShapeDtypeStruct((npad,), jnp.float32),
                  jax.ShapeDtypeStruct((npad,), jnp.float32)],
        mesh=_mesh(),
        scratch_types=[
            pltpu.VMEM((WIN, L), jnp.float32),
            pltpu.VMEM((WIN, L), jnp.float32),
            pltpu.VMEM((L, WIN), jnp.float32),
            pltpu.VMEM((span,), jnp.float32),
            pltpu.VMEM((span,), jnp.float32),
            pltpu.VMEM((span,), jnp.float32),
            pltpu.VMEM((L,), jnp.float32),
            pltpu.VMEM((L,), jnp.float32),
        ],
        compiler_params=_sc_params(),
    )
    def k(a0_hbm, a1_hbm, h_hbm, dinv_hbm, b1_hbm, w2_hbm, g2_hbm, h2_hbm,
          a0_v, a1_v, hw_v, dv_v, g2_v, h2_v, b1_v, w2_v):
        c = lax.axis_index("c")
        s = lax.axis_index("s")
        wid = s * NC + c
        nb = wid * span
        pltpu.sync_copy(dinv_hbm.at[pl.ds(nb, span)], dv_v)
        pltpu.sync_copy(b1_hbm, b1_v)
        pltpu.sync_copy(w2_hbm, w2_v)

        @pl.loop(0, nwin)
        def _(w):
            pltpu.sync_copy(a0_hbm.at[pl.ds(nb + w * WIN, WIN)], a0_v)
            pltpu.sync_copy(a1_hbm.at[pl.ds(nb + w * WIN, WIN)], a1_v)
            pltpu.sync_copy(h_hbm.at[pl.ds(wid * span // 8 + w * L, L)], hw_v)
            b1r = b1_v[...]
            w2r = w2_v[...]
            for g in range(WIN // L):
                dw = dv_v[pl.ds(w * WIN + g * L, L)]
                h2g = jnp.zeros((L,), jnp.float32)
                lanes = lax.iota(jnp.int32, L)
                for u in range(L):
                    kk = g * L + u
                    di = jnp.full((L,), dw[u])
                    hrow = hw_v[kk // 8, pl.ds(16 * (kk % 8), L)]
                    o = di * (a0_v[kk, :] + a1_v[kk, :]) \
                        + di * di * hrow + b1r
                    he = jnp.where(o > 0, o, jnp.exp(o) - 1.0)
                    h2g = jnp.where(lanes == u, _lane_sum(he * w2r), h2g)
                h2_v[pl.ds(w * WIN + g * L, L)] = h2g
                g2_v[pl.ds(w * WIN + g * L, L)] = h2g * dw

        pltpu.sync_copy(g2_v, g2_hbm.at[pl.ds(nb, span)])
        pltpu.sync_copy(h2_v, h2_hbm.at[pl.ds(nb, span)])

    return k


def _sc_c(npad):
    span = npad // NW

    @functools.partial(
        pl.kernel,
        out_type=jax.ShapeDtypeStruct((npad,), jnp.float32),
        mesh=_mesh(),
        scratch_types=[
            pltpu.VMEM((span,), jnp.float32),
            pltpu.VMEM((span,), jnp.float32),
            pltpu.VMEM((span,), jnp.float32),
            pltpu.VMEM((span,), jnp.float32),
            pltpu.VMEM((span,), jnp.float32),
            pltpu.VMEM((L,), jnp.float32),
        ],
        compiler_params=_sc_params(),
    )
    def k(c0_hbm, c1_hbm, dinv_hbm, h2_hbm, b2_hbm, o_hbm,
          c0_v, c1_v, dv_v, h2_v, o_v, b2_v):
        c = lax.axis_index("c")
        s = lax.axis_index("s")
        wid = s * NC + c
        nb = wid * span
        pltpu.sync_copy(c0_hbm.at[pl.ds(nb, span)], c0_v)
        pltpu.sync_copy(c1_hbm.at[pl.ds(nb, span)], c1_v)
        pltpu.sync_copy(dinv_hbm.at[pl.ds(nb, span)], dv_v)
        pltpu.sync_copy(h2_hbm.at[pl.ds(nb, span)], h2_v)
        pltpu.sync_copy(b2_hbm, b2_v)

        @pl.loop(0, span, step=L)
        def _(i):
            sl = pl.ds(i, L)
            d = dv_v[sl]
            o_v[sl] = d * (c0_v[sl] + c1_v[sl]) + d * d * h2_v[sl] + b2_v[...]

        pltpu.sync_copy(o_v, o_hbm.at[pl.ds(nb, span)])

    return k


def kernel(x, edge_index, edge_attr, W1, b1, W2, b2):
    n = x.shape[0]
    e = edge_attr.shape[0]
    npad = ((n + NW * WIN - 1) // (NW * WIN)) * (NW * WIN)
    chunk = CHUNK_ROWS * WIN
    per_w = ((e + NW * chunk - 1) // (NW * chunk)) * chunk
    if (per_w // chunk) % 2 == 0:
        per_w += chunk
    epad = per_w * NW
    rows_w = per_w // WIN

    src = edge_index[0].astype(jnp.int32)
    dst = edge_index[1].astype(jnp.int32)
    ew = edge_attr.astype(jnp.float32)
    pad = epad - e
    src2d = jnp.concatenate([src, jnp.zeros((pad,), jnp.int32)]).reshape(-1, WIN)
    dst2d = jnp.concatenate([dst, jnp.zeros((pad,), jnp.int32)]).reshape(-1, WIN)
    ew2d = jnp.concatenate([ew, jnp.zeros((pad,), jnp.float32)]).reshape(-1, WIN)
    z1 = jnp.zeros((npad,), jnp.float32)
    xp = jnp.pad(x, ((0, npad - n), (0, 0)))
    xpack = xp.reshape(npad // 8, 200)
    wbig = jnp.kron(jnp.eye(8, dtype=jnp.float32), W1)

    deg0, deg1 = _sc_deg(npad, rows_w)(dst2d, ew2d, z1)
    h128 = _tc_h(xpack, wbig)
    g1, dinv = _sc_a2(npad)(deg0, deg1, h128)
    acc0, acc1 = _sc_edge16(npad, rows_w)(src2d, dst2d, ew2d, g1)
    g2, h2 = _sc_b(npad)(acc0, acc1, h128, dinv, b1, W2.reshape(16))
    c0, c1 = _sc_edge1(npad, rows_w)(src2d, dst2d, ew2d, g2, z1)
    out2 = _sc_c(npad)(c0, c1, dinv, h2, jnp.broadcast_to(b2, (L,)))
    return out2[:n][:, None]


# speedup vs baseline: 1.0178x; 1.0178x over previous
import dataclasses
import functools

import jax
import jax.numpy as jnp
from jax import lax
from jax.experimental import pallas as pl
from jax.experimental.pallas import tpu as pltpu
from jax.experimental.pallas import tpu_sc as plsc

NC = 2
NS = 16
NW = NC * NS
L = 16
WIN = 128
CHUNK_ROWS = 16

_mesh = lambda: plsc.VectorSubcoreMesh(core_axis_name="c", subcore_axis_name="s")


def _sc_params():
    cp = pltpu.CompilerParams()
    fields = pltpu.CompilerParams.__dataclass_fields__
    if "needs_layout_passes" in fields:
        cp = dataclasses.replace(cp, needs_layout_passes=False)
    if "use_tc_tiling_on_sc" in fields:
        cp = dataclasses.replace(cp, use_tc_tiling_on_sc=False)
    return cp


def _sc_deg(npad, rows_w):
    span = npad // NS
    nchunks = rows_w // CHUNK_ROWS

    @functools.partial(
        pl.kernel,
        out_type=[jax.ShapeDtypeStruct((npad,), jnp.float32)] * NC,
        mesh=_mesh(),
        scratch_types=[
            pltpu.VMEM((2, CHUNK_ROWS, WIN), jnp.int32),
            pltpu.VMEM((2, CHUNK_ROWS, WIN), jnp.float32),
            pltpu.VMEM((span,), jnp.float32),
            pltpu.VMEM_SHARED((npad,), jnp.float32),
            pltpu.SemaphoreType.DMA((2, 2)),
            pltpu.SemaphoreType.DMA((4,)),
        ],
        compiler_params=_sc_params(),
    )
    def k(dst_hbm, ew_hbm, z_hbm, out0_hbm, out1_hbm, dst_v, ew_v, stg_v,
          acc_sh, csem, ssem):
        c = lax.axis_index("c")
        s = lax.axis_index("s")
        wid = s * NC + c
        sp = pl.ds(s * span, span)
        pltpu.sync_copy(z_hbm.at[sp], stg_v)
        pltpu.sync_copy(stg_v, acc_sh.at[sp])
        plsc.subcore_barrier()
        base = wid * rows_w

        def issue_chunk(cc, b):
            ro = pl.ds(base + cc * CHUNK_ROWS, CHUNK_ROWS)
            pltpu.async_copy(dst_hbm.at[ro], dst_v.at[b], csem.at[0, b])
            pltpu.async_copy(ew_hbm.at[ro], ew_v.at[b], csem.at[1, b])

        def wait_chunk(b):
            ro = pl.ds(base, CHUNK_ROWS)
            pltpu.make_async_copy(dst_hbm.at[ro], dst_v.at[b],
                                  csem.at[0, b]).wait()
            pltpu.make_async_copy(ew_hbm.at[ro], ew_v.at[b],
                                  csem.at[1, b]).wait()

        def process_chunk(cc, b):
            wait_chunk(b)
            scatters = {}
            for j in range(CHUNK_ROWS):
                if j - 4 in scatters:
                    scatters[j - 4].wait()
                scatters[j] = pltpu.async_copy(
                    ew_v.at[b, j], acc_sh.at[dst_v.at[b, j]], ssem.at[j % 4],
                    add=True)
            for j in range(CHUNK_ROWS - 4, CHUNK_ROWS):
                scatters[j].wait()

            @pl.when(cc + 2 < nchunks)
            def _():
                issue_chunk(cc + 2, b)

        issue_chunk(0, 0)
        issue_chunk(1, 1)

        @pl.loop(0, nchunks - 1, step=2)
        def _(cc):
            process_chunk(cc, 0)
            process_chunk(cc + 1, 1)

        process_chunk(nchunks - 1, 0)

        plsc.subcore_barrier()
        pltpu.sync_copy(acc_sh.at[sp], stg_v)

        @pl.when(c == 0)
        def _():
            pltpu.sync_copy(stg_v, out0_hbm.at[sp])

        @pl.when(c == 1)
        def _():
            pltpu.sync_copy(stg_v, out1_hbm.at[sp])

    return k


def _sc_edge16(npad, rows_w):
    span = npad // NS
    nchunks = rows_w // CHUNK_ROWS

    @functools.partial(
        pl.kernel,
        out_type=[jax.ShapeDtypeStruct((npad, L), jnp.float32)] * NC,
        mesh=_mesh(),
        scratch_types=[
            pltpu.VMEM((2, CHUNK_ROWS, WIN), jnp.int32),
            pltpu.VMEM((2, CHUNK_ROWS, WIN), jnp.int32),
            pltpu.VMEM((2, CHUNK_ROWS, WIN), jnp.float32),
            pltpu.VMEM((2, WIN, L), jnp.float32),
            pltpu.VMEM((2, WIN, L), jnp.float32),
            pltpu.VMEM_SHARED((npad, L), jnp.float32),
            pltpu.SemaphoreType.DMA((3, 2)),
            pltpu.SemaphoreType.DMA((2,)),
            pltpu.SemaphoreType.DMA((2,)),
        ],
        compiler_params=_sc_params(),
    )
    def k(src_hbm, dst_hbm, ew_hbm, g1_hbm, out0_hbm, out1_hbm,
          src_v, dst_v, ew_v, rows_v, pay_v, acc_sh, csem, gsem, ssem):
        c = lax.axis_index("c")
        s = lax.axis_index("s")
        wid = s * NC + c

        @pl.loop(0, WIN)
        def _(i):
            pay_v[0, i, :] = jnp.zeros((L,), jnp.float32)

        @pl.loop(0, span, step=WIN)
        def _(r):
            pltpu.sync_copy(pay_v.at[0], acc_sh.at[pl.ds(s * span + r, WIN)])

        plsc.subcore_barrier()
        base = wid * rows_w

        def issue_chunk(cc, b):
            ro = pl.ds(base + cc * CHUNK_ROWS, CHUNK_ROWS)
            pltpu.async_copy(src_hbm.at[ro], src_v.at[b], csem.at[0, b])
            pltpu.async_copy(dst_hbm.at[ro], dst_v.at[b], csem.at[1, b])
            pltpu.async_copy(ew_hbm.at[ro], ew_v.at[b], csem.at[2, b])

        def wait_chunk(b):
            ro = pl.ds(base, CHUNK_ROWS)
            pltpu.make_async_copy(src_hbm.at[ro], src_v.at[b],
                                  csem.at[0, b]).wait()
            pltpu.make_async_copy(dst_hbm.at[ro], dst_v.at[b],
                                  csem.at[1, b]).wait()
            pltpu.make_async_copy(ew_hbm.at[ro], ew_v.at[b],
                                  csem.at[2, b]).wait()

        def scale_window(b, j, jb):
            @pl.loop(0, WIN // L)
            def _(t):
                col = pl.multiple_of(t * L, L)
                w16 = ew_v[b, j, pl.ds(col, L)]
                for u in range(L):
                    i = t * L + u
                    wv = jnp.full((L,), w16[u])
                    pay_v[jb, i, :] = rows_v[jb, i, :] * wv

        def process_chunk(cc, b):
            wait_chunk(b)
            gathers = {}
            scatters = {}
            gathers[0] = pltpu.async_copy(
                g1_hbm.at[src_v.at[b, 0]], rows_v.at[0], gsem.at[0])
            for j in range(CHUNK_ROWS):
                jb = j % 2
                if j + 1 < CHUNK_ROWS:
                    gathers[j + 1] = pltpu.async_copy(
                        g1_hbm.at[src_v.at[b, j + 1]], rows_v.at[(j + 1) % 2],
                        gsem.at[(j + 1) % 2])
                gathers[j].wait()
                if j - 2 in scatters:
                    scatters[j - 2].wait()
                scale_window(b, j, jb)
                scatters[j] = pltpu.async_copy(
                    pay_v.at[jb], acc_sh.at[dst_v.at[b, j]], ssem.at[jb],
                    add=True)
            scatters[CHUNK_ROWS - 2].wait()
            scatters[CHUNK_ROWS - 1].wait()

            @pl.when(cc + 2 < nchunks)
            def _():
                issue_chunk(cc + 2, b)

        issue_chunk(0, 0)
        issue_chunk(1, 1)

        @pl.loop(0, nchunks - 1, step=2)
        def _(cc):
            process_chunk(cc, 0)
            process_chunk(cc + 1, 1)

        process_chunk(nchunks - 1, 0)

        plsc.subcore_barrier()

        @pl.loop(0, span, step=WIN)
        def _(r):
            pltpu.sync_copy(acc_sh.at[pl.ds(s * span + r, WIN)], rows_v.at[0])

            @pl.when(c == 0)
            def _():
                pltpu.sync_copy(rows_v.at[0],
                                out0_hbm.at[pl.ds(s * span + r, WIN)])

            @pl.when(c == 1)
            def _():
                pltpu.sync_copy(rows_v.at[0],
                                out1_hbm.at[pl.ds(s * span + r, WIN)])

    return k


def _sc_edge1(npad, rows_w):
    span = npad // NS

    nchunks = rows_w // CHUNK_ROWS

    @functools.partial(
        pl.kernel,
        out_type=[jax.ShapeDtypeStruct((npad,), jnp.float32)] * NC,
        mesh=_mesh(),
        scratch_types=[
            pltpu.VMEM((2, CHUNK_ROWS, WIN), jnp.int32),
            pltpu.VMEM((2, CHUNK_ROWS, WIN), jnp.int32),
            pltpu.VMEM((2, CHUNK_ROWS, WIN), jnp.float32),
            pltpu.VMEM((4, WIN), jnp.float32),
            pltpu.VMEM((npad,), jnp.float32),
            pltpu.VMEM((span,), jnp.float32),
            pltpu.VMEM_SHARED((npad,), jnp.float32),
            pltpu.SemaphoreType.DMA((3, 2)),
            pltpu.SemaphoreType.DMA((4,)),
        ],
        compiler_params=_sc_params(),
    )
    def k(src_hbm, dst_hbm, ew_hbm, g2_hbm, z_hbm, out0_hbm, out1_hbm,
          src_v, dst_v, ew_v, pay_v, g2_v, stg_v, acc_sh, csem, ssem):
        c = lax.axis_index("c")
        s = lax.axis_index("s")
        wid = s * NC + c
        sp = pl.ds(s * span, span)
        pltpu.sync_copy(z_hbm.at[sp], stg_v)
        pltpu.sync_copy(stg_v, acc_sh.at[sp])
        pltpu.sync_copy(g2_hbm, g2_v)
        plsc.subcore_barrier()
        base = wid * rows_w

        def issue_chunk(cc, b):
            ro = pl.ds(base + cc * CHUNK_ROWS, CHUNK_ROWS)
            pltpu.async_copy(src_hbm.at[ro], src_v.at[b], csem.at[0, b])
            pltpu.async_copy(dst_hbm.at[ro], dst_v.at[b], csem.at[1, b])
            pltpu.async_copy(ew_hbm.at[ro], ew_v.at[b], csem.at[2, b])

        def wait_chunk(b):
            ro = pl.ds(base, CHUNK_ROWS)
            pltpu.make_async_copy(src_hbm.at[ro], src_v.at[b],
                                  csem.at[0, b]).wait()
            pltpu.make_async_copy(dst_hbm.at[ro], dst_v.at[b],
                                  csem.at[1, b]).wait()
            pltpu.make_async_copy(ew_hbm.at[ro], ew_v.at[b],
                                  csem.at[2, b]).wait()

        def process_chunk(cc, b):
            wait_chunk(b)
            scatters = {}
            for j in range(CHUNK_ROWS):
                jb = j % 4
                if j - 4 in scatters:
                    scatters[j - 4].wait()

                @pl.loop(0, WIN // L)
                def _(t):
                    col = pl.multiple_of(t * L, L)
                    idx16 = src_v[b, j, pl.ds(col, L)]
                    w16 = ew_v[b, j, pl.ds(col, L)]
                    vals = plsc.load_gather(g2_v, [idx16])
                    pay_v[jb, pl.ds(col, L)] = vals * w16

                scatters[j] = pltpu.async_copy(
                    pay_v.at[jb], acc_sh.at[dst_v.at[b, j]], ssem.at[jb],
                    add=True)
            for j in range(CHUNK_ROWS - 4, CHUNK_ROWS):
                scatters[j].wait()

            @pl.when(cc + 2 < nchunks)
            def _():
                issue_chunk(cc + 2, b)

        issue_chunk(0, 0)
        issue_chunk(1, 1)

        @pl.loop(0, nchunks - 1, step=2)
        def _(cc):
            process_chunk(cc, 0)
            process_chunk(cc + 1, 1)

        process_chunk(nchunks - 1, 0)

        plsc.subcore_barrier()
        pltpu.sync_copy(acc_sh.at[sp], stg_v)

        @pl.when(c == 0)
        def _():
            pltpu.sync_copy(stg_v, out0_hbm.at[sp])

        @pl.when(c == 1)
        def _():
            pltpu.sync_copy(stg_v, out1_hbm.at[sp])

    return k


_BLK = 2048


def _row_spec(cols):
    return pl.BlockSpec((_BLK, cols), lambda i: (i, 0))


def _full_spec(shape):
    return pl.BlockSpec(shape, lambda i: tuple(0 for _ in shape))


def _tc_h(xpack, wbig):
    rows = xpack.shape[0]
    blk = _BLK // 8

    def body(x_ref, w_ref, h_ref):
        h_ref[...] = jnp.dot(x_ref[...], w_ref[...],
                             preferred_element_type=jnp.float32)

    return pl.pallas_call(
        body,
        grid=(rows // blk,),
        in_specs=[pl.BlockSpec((blk, 200), lambda i: (i, 0)),
                  _full_spec((200, 128))],
        out_specs=pl.BlockSpec((blk, 128), lambda i: (i, 0)),
        out_shape=jax.ShapeDtypeStruct((rows, 128), jnp.float32),
    )(xpack, wbig)


def _rsqrt16(x):
    i = plsc.bitcast(x, jnp.int32)
    i = jnp.int32(0x5F3759DF) - (i >> 1)
    y = plsc.bitcast(i, jnp.float32)
    for _ in range(3):
        y = y * (1.5 - 0.5 * x * y * y)
    return y


def _sc_a2(npad):
    span = npad // NW
    nwin = span // WIN

    @functools.partial(
        pl.kernel,
        out_type=[jax.ShapeDtypeStruct((npad, L), jnp.float32),
                  jax.ShapeDtypeStruct((npad,), jnp.float32)],
        mesh=_mesh(),
        scratch_types=[
            pltpu.VMEM((span,), jnp.float32),
            pltpu.VMEM((span,), jnp.float32),
            pltpu.VMEM((span,), jnp.float32),
            pltpu.VMEM((L, WIN), jnp.float32),
            pltpu.VMEM((WIN, L), jnp.float32),
        ],
        compiler_params=_sc_params(),
    )
    def k(d0_hbm, d1_hbm, h_hbm, g1_hbm, dinv_hbm, d0_v, d1_v, dv_v, hw_v,
          gw_v):
        c = lax.axis_index("c")
        s = lax.axis_index("s")
        wid = s * NC + c
        nb = wid * span
        pltpu.sync_copy(d0_hbm.at[pl.ds(nb, span)], d0_v)
        pltpu.sync_copy(d1_hbm.at[pl.ds(nb, span)], d1_v)

        @pl.loop(0, span, step=L)
        def _(i):
            d = d0_v[pl.ds(i, L)] + d1_v[pl.ds(i, L)] + 1.0
            dv_v[pl.ds(i, L)] = _rsqrt16(d)

        pltpu.sync_copy(dv_v, dinv_hbm.at[pl.ds(nb, span)])

        @pl.loop(0, nwin)
        def _(w):
            pltpu.sync_copy(h_hbm.at[pl.ds(wid * span // 8 + w * L, L)], hw_v)
            for g in range(WIN // L):
                dw = dv_v[pl.ds(w * WIN + g * L, L)]
                for u in range(L):
                    kk = g * L + u
                    gw_v[kk, :] = hw_v[kk // 8, pl.ds(16 * (kk % 8), L)] \
                        * jnp.full((L,), dw[u])
            pltpu.sync_copy(gw_v, g1_hbm.at[pl.ds(nb + w * WIN, WIN)])

    return k


_GATHER_DNUMS = lax.GatherDimensionNumbers(
    offset_dims=(), collapsed_slice_dims=(0,), start_index_map=(0,))


def _lane_perm(v, idx):
    return lax.gather(v, idx[:, None], _GATHER_DNUMS, slice_sizes=(1,),
                      mode=lax.GatherScatterMode.PROMISE_IN_BOUNDS)


def _lane_sum(v):
    lanes = lax.iota(jnp.int32, L)
    for k in (1, 2, 4, 8):
        v = v + _lane_perm(v, lanes ^ k)
    return v


def _sc_b(npad):
    span = npad // NW
    nwin = span // WIN

    @functools.partial(
        pl.kernel,
        out_type=[jax.ShapeDtypeStruct((npad,), jnp.float32),
                  jax.ShapeDtypeStruct((npad,), jnp.float32)],
        mesh=_mesh(),
        scratch_types=[
            pltpu.VMEM((WIN, L), jnp.float32),
            pltpu.VMEM((WIN, L), jnp.float32),
            pltpu.VMEM((L, WIN), jnp.float32),
            pltpu.VMEM((span,), jnp.float32),
            pltpu.VMEM((span,), jnp.float32),
            pltpu.VMEM((span,), jnp.float32),
            pltpu.VMEM((L,), jnp.float32),
            pltpu.VMEM((L,), jnp.float32),
        ],
        compiler_params=_sc_params(),
    )
    def k(a0_hbm, a1_hbm, h_hbm, dinv_hbm, b1_hbm, w2_hbm, g2_hbm, h2_hbm,
          a0_v, a1_v, hw_v, dv_v, g2_v, h2_v, b1_v, w2_v):
        c = lax.axis_index("c")
        s = lax.axis_index("s")
        wid = s * NC + c
        nb = wid * span
        pltpu.sync_copy(dinv_hbm.at[pl.ds(nb, span)], dv_v)
        pltpu.sync_copy(b1_hbm, b1_v)
        pltpu.sync_copy(w2_hbm, w2_v)

        @pl.loop(0, nwin)
        def _(w):
            pltpu.sync_copy(a0_hbm.at[pl.ds(nb + w * WIN, WIN)], a0_v)
            pltpu.sync_copy(a1_hbm.at[pl.ds(nb + w * WIN, WIN)], a1_v)
            pltpu.sync_copy(h_hbm.at[pl.ds(wid * span // 8 + w * L, L)], hw_v)
            b1r = b1_v[...]
            w2r = w2_v[...]
            for g in range(WIN // L):
                dw = dv_v[pl.ds(w * WIN + g * L, L)]
                h2g = jnp.zeros((L,), jnp.float32)
                lanes = lax.iota(jnp.int32, L)
                for u in range(L):
                    kk = g * L + u
                    di = jnp.full((L,), dw[u])
                    hrow = hw_v[kk // 8, pl.ds(16 * (kk % 8), L)]
                    o = di * (a0_v[kk, :] + a1_v[kk, :]) \
                        + di * di * hrow + b1r
                    he = jnp.where(o > 0, o, jnp.exp(o) - 1.0)
                    h2g = jnp.where(lanes == u, _lane_sum(he * w2r), h2g)
                h2_v[pl.ds(w * WIN + g * L, L)] = h2g
                g2_v[pl.ds(w * WIN + g * L, L)] = h2g * dw

        pltpu.sync_copy(g2_v, g2_hbm.at[pl.ds(nb, span)])
        pltpu.sync_copy(h2_v, h2_hbm.at[pl.ds(nb, span)])

    return k


def _sc_c(npad):
    span = npad // NW

    @functools.partial(
        pl.kernel,
        out_type=jax.ShapeDtypeStruct((npad,), jnp.float32),
        mesh=_mesh(),
        scratch_types=[
            pltpu.VMEM((span,), jnp.float32),
            pltpu.VMEM((span,), jnp.float32),
            pltpu.VMEM((span,), jnp.float32),
            pltpu.VMEM((span,), jnp.float32),
            pltpu.VMEM((span,), jnp.float32),
            pltpu.VMEM((L,), jnp.float32),
        ],
        compiler_params=_sc_params(),
    )
    def k(c0_hbm, c1_hbm, dinv_hbm, h2_hbm, b2_hbm, o_hbm,
          c0_v, c1_v, dv_v, h2_v, o_v, b2_v):
        c = lax.axis_index("c")
        s = lax.axis_index("s")
        wid = s * NC + c
        nb = wid * span
        pltpu.sync_copy(c0_hbm.at[pl.ds(nb, span)], c0_v)
        pltpu.sync_copy(c1_hbm.at[pl.ds(nb, span)], c1_v)
        pltpu.sync_copy(dinv_hbm.at[pl.ds(nb, span)], dv_v)
        pltpu.sync_copy(h2_hbm.at[pl.ds(nb, span)], h2_v)
        pltpu.sync_copy(b2_hbm, b2_v)

        @pl.loop(0, span, step=L)
        def _(i):
            sl = pl.ds(i, L)
            d = dv_v[sl]
            o_v[sl] = d * (c0_v[sl] + c1_v[sl]) + d * d * h2_v[sl] + b2_v[...]

        pltpu.sync_copy(o_v, o_hbm.at[pl.ds(nb, span)])

    return k


def kernel(x, edge_index, edge_attr, W1, b1, W2, b2):
    n = x.shape[0]
    e = edge_attr.shape[0]
    npad = ((n + NW * WIN - 1) // (NW * WIN)) * (NW * WIN)
    chunk = CHUNK_ROWS * WIN
    per_w = ((e + NW * chunk - 1) // (NW * chunk)) * chunk
    if (per_w // chunk) % 2 == 0:
        per_w += chunk
    epad = per_w * NW
    rows_w = per_w // WIN

    src = edge_index[0].astype(jnp.int32)
    dst = edge_index[1].astype(jnp.int32)
    ew = edge_attr.astype(jnp.float32)
    pad = epad - e
    src2d = jnp.concatenate([src, jnp.zeros((pad,), jnp.int32)]).reshape(-1, WIN)
    dst2d = jnp.concatenate([dst, jnp.zeros((pad,), jnp.int32)]).reshape(-1, WIN)
    ew2d = jnp.concatenate([ew, jnp.zeros((pad,), jnp.float32)]).reshape(-1, WIN)
    z1 = jnp.zeros((npad,), jnp.float32)
    xp = jnp.pad(x, ((0, npad - n), (0, 0)))
    xpack = xp.reshape(npad // 8, 200)
    wbig = jnp.kron(jnp.eye(8, dtype=jnp.float32), W1)

    deg0, deg1 = _sc_deg(npad, rows_w)(dst2d, ew2d, z1)
    h128 = _tc_h(xpack, wbig)
    g1, dinv = _sc_a2(npad)(deg0, deg1, h128)
    acc0, acc1 = _sc_edge16(npad, rows_w)(src2d, dst2d, ew2d, g1)
    g2, h2 = _sc_b(npad)(acc0, acc1, h128, dinv, b1, W2.reshape(16))
    c0, c1 = _sc_edge1(npad, rows_w)(src2d, dst2d, ew2d, g2, z1)
    out2 = _sc_c(npad)(c0, c1, dinv, h2, jnp.broadcast_to(b2, (L,)))
    return out2[:n][:, None]
